# Initial kernel scaffold; baseline (speedup 1.0000x reference)
import jax
import jax.numpy as jnp
from jax.experimental import pallas as pl


def kernel(prob, target, weight):
    raise NotImplementedError("write your pallas kernel here")


# speedup vs baseline: 1.7143x; 1.7143x over previous
import functools

import jax
import jax.numpy as jnp
from jax import lax
from jax.experimental import pallas as pl
from jax.experimental.pallas import tpu as pltpu
from jax.experimental.pallas import tpu_sc as plsc

LANES = 16
CHUNK = 128


@functools.lru_cache(maxsize=None)
def _make_nll_kernel(n: int, c: int):
    info = plsc.get_sparse_core_info()
    nc, ns = info.num_cores, info.num_subcores
    nw = nc * ns
    per_w = n // nw
    n_chunks = per_w // CHUNK
    n_steps = per_w // LANES
    steps_per_chunk = CHUNK // LANES

    mesh = plsc.VectorSubcoreMesh(core_axis_name="c", subcore_axis_name="s")

    @functools.partial(
        pl.kernel,
        mesh=mesh,
        out_type=jax.ShapeDtypeStruct((nw, LANES), jnp.float32),
        compiler_params=pltpu.CompilerParams(needs_layout_passes=False),
        scratch_types=[
            pltpu.VMEM((per_w,), jnp.int32),
            pltpu.VMEM((c,), jnp.float32),
            pltpu.VMEM((per_w,), jnp.int32),
            pltpu.VMEM((per_w,), jnp.float32),
            pltpu.VMEM((LANES,), jnp.float32),
            pltpu.SemaphoreType.DMA,
        ],
    )
    def nll(pflat_hbm, tgt_hbm, w_hbm, out_hbm, t_v, w_v, idx_v, val_v, res_v, sem):
        cid = lax.axis_index("c")
        sid = lax.axis_index("s")
        wid = sid * nc + cid
        base = wid * per_w

        pltpu.sync_copy(tgt_hbm.at[pl.ds(base, per_w)], t_v)
        pltpu.sync_copy(w_hbm, w_v)

        lane_off = lax.iota(jnp.int32, LANES) * c
        base_off = base * c
        for j in range(n_steps):
            t16 = t_v[pl.ds(j * LANES, LANES)]
            idx_v[pl.ds(j * LANES, LANES)] = t16 + (lane_off + (base_off + j * LANES * c))

        copies = []
        for ci in range(n_chunks):
            sl = pl.ds(ci * CHUNK, CHUNK)
            copies.append(pltpu.async_copy(pflat_hbm.at[idx_v.at[sl]], val_v.at[sl], sem))
        for cp in copies:
            cp.wait()

        acc = jnp.zeros((LANES,), jnp.float32)
        for j in range(n_steps):
            sl = pl.ds(j * LANES, LANES)
            wsel = plsc.load_gather(w_v, [t_v[sl]])
            acc = acc + val_v[sl] * wsel

        res_v[...] = jnp.full((LANES,), jnp.sum(acc), jnp.float32)
        pltpu.sync_copy(res_v, out_hbm.at[wid])

    return nll


def kernel(prob, target, weight):
    n, c = prob.shape
    pflat = prob.reshape(n * c)
    tgt = target.astype(jnp.int32)
    partials = _make_nll_kernel(n, c)(pflat, tgt, weight.astype(jnp.float32))
    return -jnp.sum(partials[:, 0])


# speedup vs baseline: 1.8179x; 1.0604x over previous
import functools

import jax
import jax.numpy as jnp
from jax import lax
from jax.experimental import pallas as pl
from jax.experimental.pallas import tpu as pltpu
from jax.experimental.pallas import tpu_sc as plsc

LANES = 16
CHUNK = 128


@functools.lru_cache(maxsize=None)
def _make_nll_kernel(n: int, c: int):
    info = plsc.get_sparse_core_info()
    nc, ns = info.num_cores, info.num_subcores
    nw = nc * ns
    per_w = n // nw
    n_chunks = per_w // CHUNK
    n_steps = per_w // LANES
    steps_per_chunk = CHUNK // LANES

    mesh = plsc.VectorSubcoreMesh(core_axis_name="c", subcore_axis_name="s")

    @functools.partial(
        pl.kernel,
        mesh=mesh,
        out_type=jax.ShapeDtypeStruct((nw, LANES), jnp.float32),
        compiler_params=pltpu.CompilerParams(needs_layout_passes=False),
        scratch_types=[
            pltpu.VMEM((per_w,), jnp.int32),
            pltpu.VMEM((c,), jnp.float32),
            pltpu.VMEM((per_w,), jnp.int32),
            pltpu.VMEM((per_w,), jnp.float32),
            pltpu.VMEM((LANES,), jnp.float32),
            pltpu.SemaphoreType.DMA,
        ],
    )
    def nll(pflat_hbm, tgt_hbm, w_hbm, out_hbm, t_v, w_v, idx_v, val_v, res_v, sem):
        cid = lax.axis_index("c")
        sid = lax.axis_index("s")
        wid = sid * nc + cid
        base = wid * per_w

        pltpu.sync_copy(tgt_hbm.at[pl.ds(base, per_w)], t_v)
        pltpu.sync_copy(w_hbm, w_v)

        lane_base = lax.iota(jnp.int32, LANES) * c + base * c

        def fire(ci, carry):
            off = ci * CHUNK
            for k in range(CHUNK // LANES):
                sl = pl.ds(off + k * LANES, LANES)
                idx_v[sl] = t_v[sl] + (lane_base + (off + k * LANES) * c)
            sl_c = pl.ds(off, CHUNK)
            pltpu.async_copy(pflat_hbm.at[idx_v.at[sl_c]], val_v.at[sl_c], sem)
            return carry

        lax.fori_loop(0, n_chunks, fire, 0)

        pltpu.make_async_copy(pflat_hbm.at[pl.ds(0, per_w)], val_v, sem).wait()

        def accum(ci, acc):
            off = ci * CHUNK
            for k in range(CHUNK // LANES):
                sl = pl.ds(off + k * LANES, LANES)
                acc = acc + val_v[sl] * plsc.load_gather(w_v, [t_v[sl]])
            return acc

        acc = lax.fori_loop(0, n_chunks, accum, jnp.zeros((LANES,), jnp.float32))

        res_v[...] = jnp.full((LANES,), jnp.sum(acc), jnp.float32)
        pltpu.sync_copy(res_v, out_hbm.at[wid])

    return nll


def kernel(prob, target, weight):
    n, c = prob.shape
    pflat = prob.reshape(n * c)
    tgt = target.astype(jnp.int32)
    partials = _make_nll_kernel(n, c)(pflat, tgt, weight.astype(jnp.float32))
    return -jnp.sum(partials[:, 0])


# speedup vs baseline: 1.8580x; 1.0221x over previous
import functools

import jax
import jax.numpy as jnp
from jax import lax
from jax.experimental import pallas as pl
from jax.experimental.pallas import tpu as pltpu
from jax.experimental.pallas import tpu_sc as plsc

LANES = 16
CHUNK = 128


@functools.lru_cache(maxsize=None)
def _make_nll_kernel(n: int, c: int):
    info = plsc.get_sparse_core_info()
    nc, ns = info.num_cores, info.num_subcores
    nw = nc * ns
    per_w = n // nw
    n_chunks = per_w // CHUNK
    n_steps = per_w // LANES
    steps_per_chunk = CHUNK // LANES

    mesh = plsc.VectorSubcoreMesh(core_axis_name="c", subcore_axis_name="s")

    @functools.partial(
        pl.kernel,
        mesh=mesh,
        out_type=jax.ShapeDtypeStruct((nw, LANES), jnp.float32),
        compiler_params=pltpu.CompilerParams(needs_layout_passes=False),
        scratch_types=[
            pltpu.VMEM((per_w,), jnp.int32),
            pltpu.VMEM((c,), jnp.float32),
            pltpu.VMEM((per_w,), jnp.int32),
            pltpu.VMEM((per_w,), jnp.float32),
            pltpu.VMEM((LANES,), jnp.float32),
            pltpu.SemaphoreType.DMA((n_chunks,)),
        ],
    )
    def nll(pflat_hbm, tgt_hbm, w_hbm, out_hbm, t_v, w_v, idx_v, val_v, res_v, sem):
        cid = lax.axis_index("c")
        sid = lax.axis_index("s")
        wid = sid * nc + cid
        base = wid * per_w

        pltpu.sync_copy(tgt_hbm.at[pl.ds(base, per_w)], t_v)
        pltpu.sync_copy(w_hbm, w_v)

        lane_base = lax.iota(jnp.int32, LANES) * c + base * c

        def fire(ci, carry):
            off = ci * CHUNK
            for k in range(CHUNK // LANES):
                sl = pl.ds(off + k * LANES, LANES)
                idx_v[sl] = t_v[sl] + (lane_base + (off + k * LANES) * c)
            sl_c = pl.ds(off, CHUNK)
            pltpu.async_copy(pflat_hbm.at[idx_v.at[sl_c]], val_v.at[sl_c], sem.at[ci])
            return carry

        lax.fori_loop(0, n_chunks, fire, 0)

        def accum(ci, acc):
            off = ci * CHUNK
            sl_c = pl.ds(off, CHUNK)
            pltpu.make_async_copy(pflat_hbm.at[pl.ds(0, CHUNK)], val_v.at[sl_c], sem.at[ci]).wait()
            for k in range(CHUNK // LANES):
                sl = pl.ds(off + k * LANES, LANES)
                acc = acc + val_v[sl] * plsc.load_gather(w_v, [t_v[sl]])
            return acc

        acc = lax.fori_loop(0, n_chunks, accum, jnp.zeros((LANES,), jnp.float32))

        res_v[...] = jnp.full((LANES,), jnp.sum(acc), jnp.float32)
        pltpu.sync_copy(res_v, out_hbm.at[wid])

    return nll


def kernel(prob, target, weight):
    n, c = prob.shape
    pflat = prob.reshape(n * c)
    tgt = target.astype(jnp.int32)
    partials = _make_nll_kernel(n, c)(pflat, tgt, weight.astype(jnp.float32))
    return -jnp.sum(partials[:, 0])


# speedup vs baseline: 1.8615x; 1.0019x over previous
import functools

import jax
import jax.numpy as jnp
from jax import lax
from jax.experimental import pallas as pl
from jax.experimental.pallas import tpu as pltpu
from jax.experimental.pallas import tpu_sc as plsc

LANES = 16
CHUNK = 128


@functools.lru_cache(maxsize=None)
def _make_nll_kernel(n: int, c: int):
    info = plsc.get_sparse_core_info()
    nc, ns = info.num_cores, info.num_subcores
    nw = nc * ns
    per_w = n // nw
    n_chunks = per_w // CHUNK
    n_steps = per_w // LANES
    steps_per_chunk = CHUNK // LANES

    mesh = plsc.VectorSubcoreMesh(core_axis_name="c", subcore_axis_name="s")

    @functools.partial(
        pl.kernel,
        mesh=mesh,
        out_type=jax.ShapeDtypeStruct((nw, LANES), jnp.float32),
        compiler_params=pltpu.CompilerParams(
            needs_layout_passes=False, skip_device_barrier=True
        ),
        scratch_types=[
            pltpu.VMEM((per_w,), jnp.int32),
            pltpu.VMEM((c,), jnp.float32),
            pltpu.VMEM((per_w,), jnp.int32),
            pltpu.VMEM((per_w,), jnp.float32),
            pltpu.VMEM((LANES,), jnp.float32),
            pltpu.SemaphoreType.DMA((n_chunks,)),
        ],
    )
    def nll(pflat_hbm, tgt_hbm, w_hbm, out_hbm, t_v, w_v, idx_v, val_v, res_v, sem):
        cid = lax.axis_index("c")
        sid = lax.axis_index("s")
        wid = sid * nc + cid
        base = wid * per_w

        pltpu.sync_copy(tgt_hbm.at[pl.ds(base, per_w)], t_v)
        pltpu.sync_copy(w_hbm, w_v)

        lane_base = lax.iota(jnp.int32, LANES) * c + base * c

        def fire(ci, carry):
            off = ci * CHUNK
            for k in range(CHUNK // LANES):
                sl = pl.ds(off + k * LANES, LANES)
                idx_v[sl] = t_v[sl] + (lane_base + (off + k * LANES) * c)
            sl_c = pl.ds(off, CHUNK)
            pltpu.async_copy(pflat_hbm.at[idx_v.at[sl_c]], val_v.at[sl_c], sem.at[ci])
            return carry

        lax.fori_loop(0, n_chunks, fire, 0)

        def accum(ci, acc):
            off = ci * CHUNK
            sl_c = pl.ds(off, CHUNK)
            pltpu.make_async_copy(pflat_hbm.at[pl.ds(0, CHUNK)], val_v.at[sl_c], sem.at[ci]).wait()
            for k in range(CHUNK // LANES):
                sl = pl.ds(off + k * LANES, LANES)
                acc = acc + val_v[sl] * plsc.load_gather(w_v, [t_v[sl]])
            return acc

        acc = lax.fori_loop(0, n_chunks, accum, jnp.zeros((LANES,), jnp.float32))

        res_v[...] = jnp.full((LANES,), jnp.sum(acc), jnp.float32)
        pltpu.sync_copy(res_v, out_hbm.at[wid])

    return nll


def kernel(prob, target, weight):
    n, c = prob.shape
    pflat = prob.reshape(n * c)
    tgt = target.astype(jnp.int32)
    partials = _make_nll_kernel(n, c)(pflat, tgt, weight.astype(jnp.float32))
    return -jnp.sum(partials[:, 0])
